# baseline (speedup 1.0000x reference)
import functools

import jax
import jax.numpy as jnp
from jax import lax
from jax.experimental import pallas as pl
from jax.experimental.pallas import tpu as pltpu
from jax.experimental.pallas import tpu_sc as plsc

DIM = 128
MP_STEPS = 3
N_NODES = 10000
N_EDGES = 320000

NC, NS, L = 2, 16, 16
NW = NC * NS
CHUNK = 128
CPT = 80
E_PAD = NW * CPT * CHUNK
TRASH = N_NODES
AGG_ROWS = 10112

NODE_BLK = 400
EDGE_BLK = 2048

_SC_MESH = plsc.VectorSubcoreMesh(core_axis_name="c", subcore_axis_name="s")


HALF = DIM // 2


def _pack_i32(x):
    xb = jax.lax.bitcast_convert_type(x.astype(jnp.bfloat16), jnp.uint16)
    lo = xb[:, :HALF].astype(jnp.uint32)
    hi = xb[:, HALF:].astype(jnp.uint32)
    return jax.lax.bitcast_convert_type(lo | (hi << 16), jnp.int32)


def _unpack_f32(w):
    lo = jax.lax.bitcast_convert_type(w << 16, jnp.float32)
    hi = jax.lax.bitcast_convert_type(
        w & jnp.int32(-65536), jnp.float32)
    return jnp.concatenate([lo, hi], axis=-1)


def _b12_body(h_ref, pos_ref, w1_ref, w2_ref, w3_ref, b_ref, b1_ref, b2_ref):
    h = h_ref[...]
    p = pos_ref[...] @ w3_ref[...]
    b1_ref[...] = h @ w1_ref[...] + p
    b2_ref[...] = h @ w2_ref[...] - p + b_ref[...]


def _compute_b12(h, pos, w1, w2, w3, b):
    return pl.pallas_call(
        _b12_body,
        grid=(N_NODES // NODE_BLK,),
        in_specs=[
            pl.BlockSpec((NODE_BLK, DIM), lambda i: (i, 0)),
            pl.BlockSpec((NODE_BLK, 8), lambda i: (i, 0)),
            pl.BlockSpec((DIM, DIM), lambda i: (0, 0)),
            pl.BlockSpec((DIM, DIM), lambda i: (0, 0)),
            pl.BlockSpec((8, DIM), lambda i: (0, 0)),
            pl.BlockSpec((1, DIM), lambda i: (0, 0)),
        ],
        out_specs=[
            pl.BlockSpec((NODE_BLK, DIM), lambda i: (i, 0)),
            pl.BlockSpec((NODE_BLK, DIM), lambda i: (i, 0)),
        ],
        out_shape=[
            jax.ShapeDtypeStruct((N_NODES, DIM), jnp.float32),
            jax.ShapeDtypeStruct((N_NODES, DIM), jnp.float32),
        ],
    )(h, pos, w1, w2, w3, b)


def _ln(x, g, b, eps=1e-5):
    mu = jnp.mean(x, axis=-1, keepdims=True)
    var = jnp.mean((x - mu) ** 2, axis=-1, keepdims=True)
    return (x - mu) * jax.lax.rsqrt(var + eps) * g + b


def _edge_body(t_ref, g_ref, bt_ref, m_ref):
    t = t_ref[...]
    m = t * 0.5 * (1.0 + jax.lax.erf(t * 0.7071067811865476))
    m_ref[...] = _ln(m, g_ref[...], bt_ref[...])


def _edge_mlp(t, g, bt):
    n_rows = t.shape[0]
    return pl.pallas_call(
        _edge_body,
        grid=(n_rows // EDGE_BLK,),
        in_specs=[
            pl.BlockSpec((EDGE_BLK, DIM), lambda i: (i, 0)),
            pl.BlockSpec((1, DIM), lambda i: (0, 0)),
            pl.BlockSpec((1, DIM), lambda i: (0, 0)),
        ],
        out_specs=pl.BlockSpec((EDGE_BLK, DIM), lambda i: (i, 0)),
        out_shape=jax.ShapeDtypeStruct((n_rows, DIM), jnp.float32),
    )(t, g, bt)


def _make_update(n_agg):
    node_spec = pl.BlockSpec((NODE_BLK, DIM), lambda i: (i, 0))
    cnt_spec = pl.BlockSpec((NODE_BLK, 1), lambda i: (i, 0))
    wide_spec = pl.BlockSpec((DIM, DIM), lambda i: (0, 0))
    vec_spec = pl.BlockSpec((1, DIM), lambda i: (0, 0))

    def body(*refs):
        h_ref = refs[0]
        agg_refs = refs[1:1 + n_agg]
        c0_ref, c1_ref, wu1_ref, wu2_ref, bu_ref, g_ref, bt_ref = \
            refs[1 + n_agg:8 + n_agg]
        out_ref = refs[8 + n_agg]
        h = h_ref[...]
        cnt = jnp.maximum(c0_ref[...] + c1_ref[...], 1.0)
        asum = agg_refs[0][...]
        for r in agg_refs[1:]:
            asum = asum + r[...]
        agg = asum / cnt
        u = h @ wu1_ref[...] + agg @ wu2_ref[...] + bu_ref[...]
        out_ref[...] = h + _ln(u, g_ref[...], bt_ref[...])

    def call(h, aggs, c0, c1, wu1, wu2, bu, g, bt):
        return pl.pallas_call(
            body,
            grid=(N_NODES // NODE_BLK,),
            in_specs=[node_spec] * (1 + n_agg) + [cnt_spec, cnt_spec,
                      wide_spec, wide_spec, vec_spec, vec_spec, vec_spec],
            out_specs=node_spec,
            out_shape=jax.ShapeDtypeStruct((N_NODES, DIM), jnp.float32),
        )(h, *aggs, c0, c1, wu1, wu2, bu, g, bt)

    return call


def _make_update_b12(n_agg):
    node_spec = pl.BlockSpec((NODE_BLK, DIM), lambda i: (i, 0))
    pos_spec = pl.BlockSpec((NODE_BLK, 8), lambda i: (i, 0))
    cnt_spec = pl.BlockSpec((NODE_BLK, 1), lambda i: (i, 0))
    wide_spec = pl.BlockSpec((DIM, DIM), lambda i: (0, 0))
    w3_spec = pl.BlockSpec((8, DIM), lambda i: (0, 0))
    vec_spec = pl.BlockSpec((1, DIM), lambda i: (0, 0))

    def body(*refs):
        h_ref = refs[0]
        pos_ref = refs[1]
        agg_refs = refs[2:2 + n_agg]
        (c0_ref, c1_ref, wu1_ref, wu2_ref, bu_ref, g_ref, bt_ref,
         w1_ref, w2_ref, w3_ref, bm_ref) = refs[2 + n_agg:13 + n_agg]
        out_ref, b1_ref, b2_ref = refs[13 + n_agg:]
        h = h_ref[...]
        cnt = jnp.maximum(c0_ref[...] + c1_ref[...], 1.0)
        asum = agg_refs[0][...]
        for r in agg_refs[1:]:
            asum = asum + r[...]
        agg = asum / cnt
        u = h @ wu1_ref[...] + agg @ wu2_ref[...] + bu_ref[...]
        hn = h + _ln(u, g_ref[...], bt_ref[...])
        out_ref[...] = hn
        p = pos_ref[...] @ w3_ref[...]
        b1_ref[...] = hn @ w1_ref[...] + p
        b2_ref[...] = hn @ w2_ref[...] - p + bm_ref[...]

    def call(h, pos, aggs, c0, c1, wu1, wu2, bu, g, bt, w1, w2, w3, bm):
        return pl.pallas_call(
            body,
            grid=(N_NODES // NODE_BLK,),
            in_specs=([node_spec, pos_spec] + [node_spec] * n_agg +
                      [cnt_spec, cnt_spec, wide_spec, wide_spec, vec_spec,
                       vec_spec, vec_spec, wide_spec, wide_spec, w3_spec,
                       vec_spec]),
            out_specs=[node_spec, node_spec, node_spec],
            out_shape=[
                jax.ShapeDtypeStruct((N_NODES, DIM), jnp.float32),
                jax.ShapeDtypeStruct((N_NODES, DIM), jnp.float32),
                jax.ShapeDtypeStruct((N_NODES, DIM), jnp.float32),
            ],
        )(h, pos, *aggs, c0, c1, wu1, wu2, bu, g, bt, w1, w2, w3, bm)

    return call


def _wid():
    return lax.axis_index("s") * NC + lax.axis_index("c")


def _make_gather(cpt):
    e_out = NW * cpt * CHUNK

    def body(b1_hbm, b2_hbm, row_hbm, col_hbm, t_hbm,
             ridx_all, cidx_all, r1, r2, g1sem, g2sem, wsem):
        wid = _wid()
        tile_base = pl.multiple_of(wid * (cpt * CHUNK), CHUNK)
        pltpu.sync_copy(row_hbm.at[pl.ds(wid * cpt, cpt)], ridx_all)
        pltpu.sync_copy(col_hbm.at[pl.ds(wid * cpt, cpt)], cidx_all)

        def issue_gather(s, c):
            pltpu.async_copy(b1_hbm.at[ridx_all.at[c]], r1[s], g1sem[s])
            pltpu.async_copy(b2_hbm.at[cidx_all.at[c]], r2[s], g2sem[s])

        def wait_gather(s):
            pltpu.make_async_copy(
                b1_hbm.at[pl.ds(0, CHUNK)], r1[s], g1sem[s]).wait()
            pltpu.make_async_copy(
                b2_hbm.at[pl.ds(0, CHUNK)], r2[s], g2sem[s]).wait()

        def wait_write(s):
            pltpu.make_async_copy(
                r1[s], t_hbm.at[pl.ds(0, CHUNK)], wsem[s]).wait()

        def add_and_write(s, c):
            r1s, r2s = r1[s], r2[s]

            @plsc.parallel_loop(0, CHUNK, unroll=4)
            def _(j):
                for k in range(DIM // L):
                    sl = pl.ds(k * L, L)
                    r1s[j, sl] = r1s[j, sl] + r2s[j, sl]
            base = pl.multiple_of(tile_base + c * CHUNK, CHUNK)
            pltpu.async_copy(r1[s], t_hbm.at[pl.ds(base, CHUNK)], wsem[s])

        for s in range(3):
            issue_gather(s, s)
        for s in range(3):
            wait_gather(s)
            add_and_write(s, s)
            issue_gather(s, s + 3)

        nb = (cpt - 3) // 3

        def loop_body(i, carry):
            for s in range(3):
                c = 3 * i + s
                wait_gather(s)
                wait_write(s)
                add_and_write(s, c)

                @pl.when(c + 3 < cpt)
                def _():
                    issue_gather(s, c + 3)
            return carry

        lax.fori_loop(1, 1 + nb, loop_body, 0)
        for c in range(3 + 3 * nb, cpt):
            s = c % 3
            wait_gather(s)
            wait_write(s)
            add_and_write(s, c)
        for s in range(3):
            wait_write(s)

    return functools.partial(
        pl.kernel,
        out_type=jax.ShapeDtypeStruct((e_out, DIM), jnp.float32),
        mesh=_SC_MESH,
        scratch_types=[
            pltpu.VMEM((cpt, CHUNK), jnp.int32),
            pltpu.VMEM((cpt, CHUNK), jnp.int32),
            [pltpu.VMEM((CHUNK, DIM), jnp.float32)] * 3,
            [pltpu.VMEM((CHUNK, DIM), jnp.float32)] * 3,
            [pltpu.SemaphoreType.DMA] * 3,
            [pltpu.SemaphoreType.DMA] * 3,
            [pltpu.SemaphoreType.DMA] * 3,
        ],
    )(body)


def _make_scatter(cpt):
    assert cpt % 2 == 0

    def body(m_hbm, col_hbm, zeros_hbm, out_hbm, cidx_all, mv, msem, agg):
        sid = lax.axis_index("s")
        cid = lax.axis_index("c")
        wid = _wid()
        tile_base = pl.multiple_of(wid * (cpt * CHUNK), CHUNK)
        pltpu.sync_copy(col_hbm.at[pl.ds(wid * cpt, cpt)], cidx_all)

        @pl.when(sid == 0)
        def _():
            pltpu.sync_copy(zeros_hbm, agg)

        plsc.subcore_barrier()

        def fetch_m(s, c):
            base = pl.multiple_of(tile_base + c * CHUNK, CHUNK)
            pltpu.async_copy(m_hbm.at[pl.ds(base, CHUNK)], mv[s], msem[s])

        def wait_m(s):
            pltpu.make_async_copy(
                m_hbm.at[pl.ds(0, CHUNK)], mv[s], msem[s]).wait()

        for s in range(2):
            fetch_m(s, s)

        def loop_body(i, carry):
            for s in range(2):
                c = 2 * i + s
                wait_m(s)
                pltpu.sync_copy(mv[s], agg.at[cidx_all.at[c]], add=True)

                @pl.when(c + 2 < cpt)
                def _():
                    fetch_m(s, c + 2)
            return carry

        lax.fori_loop(0, cpt // 2, loop_body, 0)
        plsc.subcore_barrier()

        @pl.when(sid == 0)
        def _():
            pltpu.sync_copy(agg, out_hbm.at[cid])

    return functools.partial(
        pl.kernel,
        out_type=jax.ShapeDtypeStruct((NC, AGG_ROWS, DIM), jnp.float32),
        mesh=_SC_MESH,
        scratch_types=[
            pltpu.VMEM((cpt, CHUNK), jnp.int32),
            [pltpu.VMEM((CHUNK, DIM), jnp.float32)] * 2,
            [pltpu.SemaphoreType.DMA] * 2,
            pltpu.VMEM_SHARED((AGG_ROWS, DIM), jnp.float32),
        ],
    )(body)


SPLITS = (24, 24, 16, 16)
_GATHERS = {c: _make_gather(c) for c in set(SPLITS)}
_SCATTERS = {c: _make_scatter(c) for c in set(SPLITS)}
N_AGG = 2 * len(SPLITS)
_upd = _make_update(N_AGG)
_upd_b12 = _make_update_b12(N_AGG)


def _cnt_body(col_hbm, zeros_hbm, out_hbm, cidx_all, ones_v, cnt):
    sid = lax.axis_index("s")
    cid = lax.axis_index("c")
    wid = _wid()
    for k in range(CHUNK // L):
        ones_v[pl.ds(k * L, L)] = jnp.full((L,), 1.0, jnp.float32)
    pltpu.sync_copy(col_hbm.at[pl.ds(wid * CPT, CPT)], cidx_all)

    @pl.when(sid == 0)
    def _():
        pltpu.sync_copy(zeros_hbm, cnt)

    plsc.subcore_barrier()

    def chunk(i, carry):
        pltpu.sync_copy(ones_v, cnt.at[cidx_all.at[i]], add=True)
        return carry

    lax.fori_loop(0, CPT, chunk, 0)
    plsc.subcore_barrier()

    @pl.when(sid == 0)
    def _():
        pltpu.sync_copy(cnt, out_hbm.at[cid])


@functools.partial(
    pl.kernel,
    out_type=jax.ShapeDtypeStruct((NC, AGG_ROWS), jnp.float32),
    mesh=_SC_MESH,
    scratch_types=[
        pltpu.VMEM((CPT, CHUNK), jnp.int32),
        pltpu.VMEM((CHUNK,), jnp.float32),
        pltpu.VMEM_SHARED((AGG_ROWS,), jnp.float32),
    ],
)
def _cnt_kernel(col_hbm, zeros_hbm, out_hbm, cidx_all, ones_v, cnt):
    _cnt_body(col_hbm, zeros_hbm, out_hbm, cidx_all, ones_v, cnt)


def kernel(x, pos, edge_index, W_msg, b_msg, g_msg, bt_msg, W_upd, b_upd,
           g_upd, bt_upd):
    row = edge_index[0].astype(jnp.int32)
    col = edge_index[1].astype(jnp.int32)
    pad = E_PAD - N_EDGES
    row_g = jnp.pad(row, (0, pad)).reshape(-1, CHUNK)
    col_g = jnp.pad(col, (0, pad)).reshape(-1, CHUNK)
    col_s = jnp.pad(col, (0, pad),
                    constant_values=TRASH).reshape(-1, CHUNK)
    pos_pad = jnp.pad(pos, ((0, 0), (0, 8 - pos.shape[1])))

    zeros2d = jnp.zeros((AGG_ROWS, DIM), jnp.float32)
    zeros1d = jnp.zeros((AGG_ROWS,), jnp.float32)

    cnt_part = _cnt_kernel(col_s, zeros1d)
    c0 = cnt_part[0][:, None]
    c1 = cnt_part[1][:, None]

    starts, accum = [], 0
    for c in SPLITS:
        starts.append(accum)
        accum += c * NW
    row_h = [row_g[s:s + c * NW] for s, c in zip(starts, SPLITS)]
    col_gh = [col_g[s:s + c * NW] for s, c in zip(starts, SPLITS)]
    col_sh = [col_s[s:s + c * NW] for s, c in zip(starts, SPLITS)]

    def msg_w(i):
        return (W_msg[i, :DIM], W_msg[i, DIM:2 * DIM],
                jnp.pad(W_msg[i, 2 * DIM:], ((0, 8 - 3), (0, 0))),
                b_msg[i][None])

    h = x
    w1, w2, w3, bm = msg_w(0)
    b1, b2 = _compute_b12(h, pos_pad, w1, w2, w3, bm)
    for i in range(MP_STEPS):
        aggs = []
        ts = [_GATHERS[c](b1, b2, row_h[k], col_gh[k])
              for k, c in enumerate(SPLITS)]
        for k, c in enumerate(SPLITS):
            m = _edge_mlp(ts[k], g_msg[i][None], bt_msg[i][None])
            agg_part = _SCATTERS[c](m, col_sh[k], zeros2d)
            aggs += [agg_part[0], agg_part[1]]
        if i + 1 < MP_STEPS:
            w1, w2, w3, bm = msg_w(i + 1)
            h, b1, b2 = _upd_b12(h, pos_pad, aggs, c0, c1,
                                 W_upd[i, :DIM], W_upd[i, DIM:],
                                 b_upd[i][None], g_upd[i][None],
                                 bt_upd[i][None], w1, w2, w3, bm)
        else:
            h = _upd(h, aggs, c0, c1,
                     W_upd[i, :DIM], W_upd[i, DIM:], b_upd[i][None],
                     g_upd[i][None], bt_upd[i][None])
    return h


# speedup vs baseline: 1.0282x; 1.0282x over previous
import functools

import jax
import jax.numpy as jnp
from jax import lax
from jax.experimental import pallas as pl
from jax.experimental.pallas import tpu as pltpu
from jax.experimental.pallas import tpu_sc as plsc

DIM = 128
MP_STEPS = 3
N_NODES = 10000
N_EDGES = 320000

NC, NS, L = 2, 16, 16
NW = NC * NS
CHUNK = 128
CPT = 80
E_PAD = NW * CPT * CHUNK
TRASH = N_NODES
AGG_ROWS = 10112

NODE_BLK = 400
EDGE_BLK = 2048

_SC_MESH = plsc.VectorSubcoreMesh(core_axis_name="c", subcore_axis_name="s")


HALF = DIM // 2


def _pack_i32(x):
    xb = jax.lax.bitcast_convert_type(x.astype(jnp.bfloat16), jnp.uint16)
    lo = xb[:, :HALF].astype(jnp.uint32)
    hi = xb[:, HALF:].astype(jnp.uint32)
    return jax.lax.bitcast_convert_type(lo | (hi << 16), jnp.int32)


def _unpack_f32(w):
    lo = jax.lax.bitcast_convert_type(w << 16, jnp.float32)
    hi = jax.lax.bitcast_convert_type(
        w & jnp.int32(-65536), jnp.float32)
    return jnp.concatenate([lo, hi], axis=-1)


def _b12_body(h_ref, pos_ref, w1_ref, w2_ref, w3_ref, b_ref, b1_ref, b2_ref):
    h = h_ref[...]
    p = pos_ref[...] @ w3_ref[...]
    b1_ref[...] = h @ w1_ref[...] + p
    b2_ref[...] = h @ w2_ref[...] - p + b_ref[...]


def _compute_b12(h, pos, w1, w2, w3, b):
    return pl.pallas_call(
        _b12_body,
        grid=(N_NODES // NODE_BLK,),
        in_specs=[
            pl.BlockSpec((NODE_BLK, DIM), lambda i: (i, 0)),
            pl.BlockSpec((NODE_BLK, 8), lambda i: (i, 0)),
            pl.BlockSpec((DIM, DIM), lambda i: (0, 0)),
            pl.BlockSpec((DIM, DIM), lambda i: (0, 0)),
            pl.BlockSpec((8, DIM), lambda i: (0, 0)),
            pl.BlockSpec((1, DIM), lambda i: (0, 0)),
        ],
        out_specs=[
            pl.BlockSpec((NODE_BLK, DIM), lambda i: (i, 0)),
            pl.BlockSpec((NODE_BLK, DIM), lambda i: (i, 0)),
        ],
        out_shape=[
            jax.ShapeDtypeStruct((N_NODES, DIM), jnp.float32),
            jax.ShapeDtypeStruct((N_NODES, DIM), jnp.float32),
        ],
    )(h, pos, w1, w2, w3, b)


def _ln(x, g, b, eps=1e-5):
    mu = jnp.mean(x, axis=-1, keepdims=True)
    var = jnp.mean((x - mu) ** 2, axis=-1, keepdims=True)
    return (x - mu) * jax.lax.rsqrt(var + eps) * g + b


def _edge_body(t_ref, g_ref, bt_ref, m_ref):
    t = t_ref[...]
    m = t * 0.5 * (1.0 + jax.lax.erf(t * 0.7071067811865476))
    m_ref[...] = _ln(m, g_ref[...], bt_ref[...])


def _edge_mlp(t, g, bt):
    n_rows = t.shape[0]
    return pl.pallas_call(
        _edge_body,
        grid=(n_rows // EDGE_BLK,),
        in_specs=[
            pl.BlockSpec((EDGE_BLK, DIM), lambda i: (i, 0)),
            pl.BlockSpec((1, DIM), lambda i: (0, 0)),
            pl.BlockSpec((1, DIM), lambda i: (0, 0)),
        ],
        out_specs=pl.BlockSpec((EDGE_BLK, DIM), lambda i: (i, 0)),
        out_shape=jax.ShapeDtypeStruct((n_rows, DIM), jnp.float32),
    )(t, g, bt)


def _make_update(n_agg):
    node_spec = pl.BlockSpec((NODE_BLK, DIM), lambda i: (i, 0))
    cnt_spec = pl.BlockSpec((NODE_BLK, 1), lambda i: (i, 0))
    wide_spec = pl.BlockSpec((DIM, DIM), lambda i: (0, 0))
    vec_spec = pl.BlockSpec((1, DIM), lambda i: (0, 0))

    def body(*refs):
        h_ref = refs[0]
        agg_refs = refs[1:1 + n_agg]
        c0_ref, c1_ref, wu1_ref, wu2_ref, bu_ref, g_ref, bt_ref = \
            refs[1 + n_agg:8 + n_agg]
        out_ref = refs[8 + n_agg]
        h = h_ref[...]
        cnt = jnp.maximum(c0_ref[...] + c1_ref[...], 1.0)
        asum = agg_refs[0][...]
        for r in agg_refs[1:]:
            asum = asum + r[...]
        agg = asum / cnt
        u = h @ wu1_ref[...] + agg @ wu2_ref[...] + bu_ref[...]
        out_ref[...] = h + _ln(u, g_ref[...], bt_ref[...])

    def call(h, aggs, c0, c1, wu1, wu2, bu, g, bt):
        return pl.pallas_call(
            body,
            grid=(N_NODES // NODE_BLK,),
            in_specs=[node_spec] * (1 + n_agg) + [cnt_spec, cnt_spec,
                      wide_spec, wide_spec, vec_spec, vec_spec, vec_spec],
            out_specs=node_spec,
            out_shape=jax.ShapeDtypeStruct((N_NODES, DIM), jnp.float32),
        )(h, *aggs, c0, c1, wu1, wu2, bu, g, bt)

    return call


def _make_update_b12(n_agg):
    node_spec = pl.BlockSpec((NODE_BLK, DIM), lambda i: (i, 0))
    pos_spec = pl.BlockSpec((NODE_BLK, 8), lambda i: (i, 0))
    cnt_spec = pl.BlockSpec((NODE_BLK, 1), lambda i: (i, 0))
    wide_spec = pl.BlockSpec((DIM, DIM), lambda i: (0, 0))
    w3_spec = pl.BlockSpec((8, DIM), lambda i: (0, 0))
    vec_spec = pl.BlockSpec((1, DIM), lambda i: (0, 0))

    def body(*refs):
        h_ref = refs[0]
        pos_ref = refs[1]
        agg_refs = refs[2:2 + n_agg]
        (c0_ref, c1_ref, wu1_ref, wu2_ref, bu_ref, g_ref, bt_ref,
         w1_ref, w2_ref, w3_ref, bm_ref) = refs[2 + n_agg:13 + n_agg]
        out_ref, b1_ref, b2_ref = refs[13 + n_agg:]
        h = h_ref[...]
        cnt = jnp.maximum(c0_ref[...] + c1_ref[...], 1.0)
        asum = agg_refs[0][...]
        for r in agg_refs[1:]:
            asum = asum + r[...]
        agg = asum / cnt
        u = h @ wu1_ref[...] + agg @ wu2_ref[...] + bu_ref[...]
        hn = h + _ln(u, g_ref[...], bt_ref[...])
        out_ref[...] = hn
        p = pos_ref[...] @ w3_ref[...]
        b1_ref[...] = hn @ w1_ref[...] + p
        b2_ref[...] = hn @ w2_ref[...] - p + bm_ref[...]

    def call(h, pos, aggs, c0, c1, wu1, wu2, bu, g, bt, w1, w2, w3, bm):
        return pl.pallas_call(
            body,
            grid=(N_NODES // NODE_BLK,),
            in_specs=([node_spec, pos_spec] + [node_spec] * n_agg +
                      [cnt_spec, cnt_spec, wide_spec, wide_spec, vec_spec,
                       vec_spec, vec_spec, wide_spec, wide_spec, w3_spec,
                       vec_spec]),
            out_specs=[node_spec, node_spec, node_spec],
            out_shape=[
                jax.ShapeDtypeStruct((N_NODES, DIM), jnp.float32),
                jax.ShapeDtypeStruct((N_NODES, DIM), jnp.float32),
                jax.ShapeDtypeStruct((N_NODES, DIM), jnp.float32),
            ],
        )(h, pos, *aggs, c0, c1, wu1, wu2, bu, g, bt, w1, w2, w3, bm)

    return call


def _wid():
    return lax.axis_index("s") * NC + lax.axis_index("c")


def _make_gather(cpt):
    e_out = NW * cpt * CHUNK

    def body(b1_hbm, b2_hbm, row_hbm, col_hbm, t_hbm,
             ridx_all, cidx_all, r1, r2, g1sem, g2sem, wsem):
        wid = _wid()
        tile_base = pl.multiple_of(wid * (cpt * CHUNK), CHUNK)
        pltpu.sync_copy(row_hbm.at[pl.ds(wid * cpt, cpt)], ridx_all)
        pltpu.sync_copy(col_hbm.at[pl.ds(wid * cpt, cpt)], cidx_all)

        def issue_gather(s, c):
            pltpu.async_copy(b1_hbm.at[ridx_all.at[c]], r1[s], g1sem[s])
            pltpu.async_copy(b2_hbm.at[cidx_all.at[c]], r2[s], g2sem[s])

        def wait_gather(s):
            pltpu.make_async_copy(
                b1_hbm.at[pl.ds(0, CHUNK)], r1[s], g1sem[s]).wait()
            pltpu.make_async_copy(
                b2_hbm.at[pl.ds(0, CHUNK)], r2[s], g2sem[s]).wait()

        def wait_write(s):
            pltpu.make_async_copy(
                r1[s], t_hbm.at[pl.ds(0, CHUNK)], wsem[s]).wait()

        def add_and_write(s, c):
            r1s, r2s = r1[s], r2[s]

            @plsc.parallel_loop(0, CHUNK, unroll=4)
            def _(j):
                for k in range(DIM // L):
                    sl = pl.ds(k * L, L)
                    r1s[j, sl] = r1s[j, sl] + r2s[j, sl]
            base = pl.multiple_of(tile_base + c * CHUNK, CHUNK)
            pltpu.async_copy(r1[s], t_hbm.at[pl.ds(base, CHUNK)], wsem[s])

        for s in range(3):
            issue_gather(s, s)
        for s in range(3):
            wait_gather(s)
            add_and_write(s, s)
            issue_gather(s, s + 3)

        nb = (cpt - 3) // 3

        def loop_body(i, carry):
            for s in range(3):
                c = 3 * i + s
                wait_gather(s)
                wait_write(s)
                add_and_write(s, c)

                @pl.when(c + 3 < cpt)
                def _():
                    issue_gather(s, c + 3)
            return carry

        lax.fori_loop(1, 1 + nb, loop_body, 0)
        for c in range(3 + 3 * nb, cpt):
            s = c % 3
            wait_gather(s)
            wait_write(s)
            add_and_write(s, c)
        for s in range(3):
            wait_write(s)

    return functools.partial(
        pl.kernel,
        out_type=jax.ShapeDtypeStruct((e_out, DIM), jnp.float32),
        mesh=_SC_MESH,
        scratch_types=[
            pltpu.VMEM((cpt, CHUNK), jnp.int32),
            pltpu.VMEM((cpt, CHUNK), jnp.int32),
            [pltpu.VMEM((CHUNK, DIM), jnp.float32)] * 3,
            [pltpu.VMEM((CHUNK, DIM), jnp.float32)] * 3,
            [pltpu.SemaphoreType.DMA] * 3,
            [pltpu.SemaphoreType.DMA] * 3,
            [pltpu.SemaphoreType.DMA] * 3,
        ],
    )(body)


def _make_scatter(cpt):
    assert cpt % 2 == 0

    def body(m_hbm, col_hbm, zeros_hbm, out_hbm, cidx_all, mv, msem, agg):
        sid = lax.axis_index("s")
        cid = lax.axis_index("c")
        wid = _wid()
        tile_base = pl.multiple_of(wid * (cpt * CHUNK), CHUNK)
        pltpu.sync_copy(col_hbm.at[pl.ds(wid * cpt, cpt)], cidx_all)

        @pl.when(sid == 0)
        def _():
            pltpu.sync_copy(zeros_hbm, agg)

        plsc.subcore_barrier()

        def fetch_m(s, c):
            base = pl.multiple_of(tile_base + c * CHUNK, CHUNK)
            pltpu.async_copy(m_hbm.at[pl.ds(base, CHUNK)], mv[s], msem[s])

        def wait_m(s):
            pltpu.make_async_copy(
                m_hbm.at[pl.ds(0, CHUNK)], mv[s], msem[s]).wait()

        for s in range(2):
            fetch_m(s, s)

        def loop_body(i, carry):
            for s in range(2):
                c = 2 * i + s
                wait_m(s)
                pltpu.sync_copy(mv[s], agg.at[cidx_all.at[c]], add=True)

                @pl.when(c + 2 < cpt)
                def _():
                    fetch_m(s, c + 2)
            return carry

        lax.fori_loop(0, cpt // 2, loop_body, 0)
        plsc.subcore_barrier()

        @pl.when(sid == 0)
        def _():
            pltpu.sync_copy(agg, out_hbm.at[cid])

    return functools.partial(
        pl.kernel,
        out_type=jax.ShapeDtypeStruct((NC, AGG_ROWS, DIM), jnp.float32),
        mesh=_SC_MESH,
        scratch_types=[
            pltpu.VMEM((cpt, CHUNK), jnp.int32),
            [pltpu.VMEM((CHUNK, DIM), jnp.float32)] * 2,
            [pltpu.SemaphoreType.DMA] * 2,
            pltpu.VMEM_SHARED((AGG_ROWS, DIM), jnp.float32),
        ],
    )(body)


SPLITS = (40, 40)
_GATHERS = {c: _make_gather(c) for c in set(SPLITS)}
_SCATTERS = {c: _make_scatter(c) for c in set(SPLITS)}
N_AGG = 2 * len(SPLITS)
_upd = _make_update(N_AGG)
_upd_b12 = _make_update_b12(N_AGG)


def _cnt_body(col_hbm, zeros_hbm, out_hbm, cidx_all, ones_v, cnt):
    sid = lax.axis_index("s")
    cid = lax.axis_index("c")
    wid = _wid()
    for k in range(CHUNK // L):
        ones_v[pl.ds(k * L, L)] = jnp.full((L,), 1.0, jnp.float32)
    pltpu.sync_copy(col_hbm.at[pl.ds(wid * CPT, CPT)], cidx_all)

    @pl.when(sid == 0)
    def _():
        pltpu.sync_copy(zeros_hbm, cnt)

    plsc.subcore_barrier()

    def chunk(i, carry):
        pltpu.sync_copy(ones_v, cnt.at[cidx_all.at[i]], add=True)
        return carry

    lax.fori_loop(0, CPT, chunk, 0)
    plsc.subcore_barrier()

    @pl.when(sid == 0)
    def _():
        pltpu.sync_copy(cnt, out_hbm.at[cid])


@functools.partial(
    pl.kernel,
    out_type=jax.ShapeDtypeStruct((NC, AGG_ROWS), jnp.float32),
    mesh=_SC_MESH,
    scratch_types=[
        pltpu.VMEM((CPT, CHUNK), jnp.int32),
        pltpu.VMEM((CHUNK,), jnp.float32),
        pltpu.VMEM_SHARED((AGG_ROWS,), jnp.float32),
    ],
)
def _cnt_kernel(col_hbm, zeros_hbm, out_hbm, cidx_all, ones_v, cnt):
    _cnt_body(col_hbm, zeros_hbm, out_hbm, cidx_all, ones_v, cnt)


def kernel(x, pos, edge_index, W_msg, b_msg, g_msg, bt_msg, W_upd, b_upd,
           g_upd, bt_upd):
    row = edge_index[0].astype(jnp.int32)
    col = edge_index[1].astype(jnp.int32)
    pad = E_PAD - N_EDGES
    row_g = jnp.pad(row, (0, pad)).reshape(-1, CHUNK)
    col_g = jnp.pad(col, (0, pad)).reshape(-1, CHUNK)
    col_s = jnp.pad(col, (0, pad),
                    constant_values=TRASH).reshape(-1, CHUNK)
    pos_pad = jnp.pad(pos, ((0, 0), (0, 8 - pos.shape[1])))

    zeros2d = jnp.zeros((AGG_ROWS, DIM), jnp.float32)
    zeros1d = jnp.zeros((AGG_ROWS,), jnp.float32)

    cnt_part = _cnt_kernel(col_s, zeros1d)
    c0 = cnt_part[0][:, None]
    c1 = cnt_part[1][:, None]

    starts, accum = [], 0
    for c in SPLITS:
        starts.append(accum)
        accum += c * NW
    row_h = [row_g[s:s + c * NW] for s, c in zip(starts, SPLITS)]
    col_gh = [col_g[s:s + c * NW] for s, c in zip(starts, SPLITS)]
    col_sh = [col_s[s:s + c * NW] for s, c in zip(starts, SPLITS)]

    def msg_w(i):
        return (W_msg[i, :DIM], W_msg[i, DIM:2 * DIM],
                jnp.pad(W_msg[i, 2 * DIM:], ((0, 8 - 3), (0, 0))),
                b_msg[i][None])

    h = x
    w1, w2, w3, bm = msg_w(0)
    b1, b2 = _compute_b12(h, pos_pad, w1, w2, w3, bm)
    for i in range(MP_STEPS):
        aggs = []
        ts = [_GATHERS[c](b1, b2, row_h[k], col_gh[k])
              for k, c in enumerate(SPLITS)]
        for k, c in enumerate(SPLITS):
            m = _edge_mlp(ts[k], g_msg[i][None], bt_msg[i][None])
            agg_part = _SCATTERS[c](m, col_sh[k], zeros2d)
            aggs += [agg_part[0], agg_part[1]]
        if i + 1 < MP_STEPS:
            w1, w2, w3, bm = msg_w(i + 1)
            h, b1, b2 = _upd_b12(h, pos_pad, aggs, c0, c1,
                                 W_upd[i, :DIM], W_upd[i, DIM:],
                                 b_upd[i][None], g_upd[i][None],
                                 bt_upd[i][None], w1, w2, w3, bm)
        else:
            h = _upd(h, aggs, c0, c1,
                     W_upd[i, :DIM], W_upd[i, DIM:], b_upd[i][None],
                     g_upd[i][None], bt_upd[i][None])
    return h


# speedup vs baseline: 1.0607x; 1.0316x over previous
import functools

import jax
import jax.numpy as jnp
from jax import lax
from jax.experimental import pallas as pl
from jax.experimental.pallas import tpu as pltpu
from jax.experimental.pallas import tpu_sc as plsc

DIM = 128
MP_STEPS = 3
N_NODES = 10000
N_EDGES = 320000

NC, NS, L = 2, 16, 16
NW = NC * NS
CHUNK = 128
CPT = 80
E_PAD = NW * CPT * CHUNK
TRASH = N_NODES
AGG_ROWS = 10112

NODE_BLK = 1000
EDGE_BLK = 4096

_SC_MESH = plsc.VectorSubcoreMesh(core_axis_name="c", subcore_axis_name="s")


HALF = DIM // 2


def _pack_i32(x):
    xb = jax.lax.bitcast_convert_type(x.astype(jnp.bfloat16), jnp.uint16)
    lo = xb[:, :HALF].astype(jnp.uint32)
    hi = xb[:, HALF:].astype(jnp.uint32)
    return jax.lax.bitcast_convert_type(lo | (hi << 16), jnp.int32)


def _unpack_f32(w):
    lo = jax.lax.bitcast_convert_type(w << 16, jnp.float32)
    hi = jax.lax.bitcast_convert_type(
        w & jnp.int32(-65536), jnp.float32)
    return jnp.concatenate([lo, hi], axis=-1)


def _b12_body(h_ref, pos_ref, w1_ref, w2_ref, w3_ref, b_ref, b1_ref, b2_ref):
    h = h_ref[...]
    p = pos_ref[...] @ w3_ref[...]
    b1_ref[...] = h @ w1_ref[...] + p
    b2_ref[...] = h @ w2_ref[...] - p + b_ref[...]


def _compute_b12(h, pos, w1, w2, w3, b):
    return pl.pallas_call(
        _b12_body,
        grid=(N_NODES // NODE_BLK,),
        in_specs=[
            pl.BlockSpec((NODE_BLK, DIM), lambda i: (i, 0)),
            pl.BlockSpec((NODE_BLK, 8), lambda i: (i, 0)),
            pl.BlockSpec((DIM, DIM), lambda i: (0, 0)),
            pl.BlockSpec((DIM, DIM), lambda i: (0, 0)),
            pl.BlockSpec((8, DIM), lambda i: (0, 0)),
            pl.BlockSpec((1, DIM), lambda i: (0, 0)),
        ],
        out_specs=[
            pl.BlockSpec((NODE_BLK, DIM), lambda i: (i, 0)),
            pl.BlockSpec((NODE_BLK, DIM), lambda i: (i, 0)),
        ],
        out_shape=[
            jax.ShapeDtypeStruct((N_NODES, DIM), jnp.float32),
            jax.ShapeDtypeStruct((N_NODES, DIM), jnp.float32),
        ],
    )(h, pos, w1, w2, w3, b)


def _ln(x, g, b, eps=1e-5):
    mu = jnp.mean(x, axis=-1, keepdims=True)
    var = jnp.mean((x - mu) ** 2, axis=-1, keepdims=True)
    return (x - mu) * jax.lax.rsqrt(var + eps) * g + b


def _edge_body(t_ref, g_ref, bt_ref, m_ref):
    t = t_ref[...]
    m = t * 0.5 * (1.0 + jax.lax.erf(t * 0.7071067811865476))
    m_ref[...] = _ln(m, g_ref[...], bt_ref[...])


def _edge_mlp(t, g, bt):
    n_rows = t.shape[0]
    return pl.pallas_call(
        _edge_body,
        grid=(n_rows // EDGE_BLK,),
        in_specs=[
            pl.BlockSpec((EDGE_BLK, DIM), lambda i: (i, 0)),
            pl.BlockSpec((1, DIM), lambda i: (0, 0)),
            pl.BlockSpec((1, DIM), lambda i: (0, 0)),
        ],
        out_specs=pl.BlockSpec((EDGE_BLK, DIM), lambda i: (i, 0)),
        out_shape=jax.ShapeDtypeStruct((n_rows, DIM), jnp.float32),
    )(t, g, bt)


def _make_update(n_agg):
    node_spec = pl.BlockSpec((NODE_BLK, DIM), lambda i: (i, 0))
    cnt_spec = pl.BlockSpec((NODE_BLK, 1), lambda i: (i, 0))
    wide_spec = pl.BlockSpec((DIM, DIM), lambda i: (0, 0))
    vec_spec = pl.BlockSpec((1, DIM), lambda i: (0, 0))

    def body(*refs):
        h_ref = refs[0]
        agg_refs = refs[1:1 + n_agg]
        c0_ref, c1_ref, wu1_ref, wu2_ref, bu_ref, g_ref, bt_ref = \
            refs[1 + n_agg:8 + n_agg]
        out_ref = refs[8 + n_agg]
        h = h_ref[...]
        cnt = jnp.maximum(c0_ref[...] + c1_ref[...], 1.0)
        asum = agg_refs[0][...]
        for r in agg_refs[1:]:
            asum = asum + r[...]
        agg = asum / cnt
        u = h @ wu1_ref[...] + agg @ wu2_ref[...] + bu_ref[...]
        out_ref[...] = h + _ln(u, g_ref[...], bt_ref[...])

    def call(h, aggs, c0, c1, wu1, wu2, bu, g, bt):
        return pl.pallas_call(
            body,
            grid=(N_NODES // NODE_BLK,),
            in_specs=[node_spec] * (1 + n_agg) + [cnt_spec, cnt_spec,
                      wide_spec, wide_spec, vec_spec, vec_spec, vec_spec],
            out_specs=node_spec,
            out_shape=jax.ShapeDtypeStruct((N_NODES, DIM), jnp.float32),
        )(h, *aggs, c0, c1, wu1, wu2, bu, g, bt)

    return call


def _make_update_b12(n_agg):
    node_spec = pl.BlockSpec((NODE_BLK, DIM), lambda i: (i, 0))
    pos_spec = pl.BlockSpec((NODE_BLK, 8), lambda i: (i, 0))
    cnt_spec = pl.BlockSpec((NODE_BLK, 1), lambda i: (i, 0))
    wide_spec = pl.BlockSpec((DIM, DIM), lambda i: (0, 0))
    w3_spec = pl.BlockSpec((8, DIM), lambda i: (0, 0))
    vec_spec = pl.BlockSpec((1, DIM), lambda i: (0, 0))

    def body(*refs):
        h_ref = refs[0]
        pos_ref = refs[1]
        agg_refs = refs[2:2 + n_agg]
        (c0_ref, c1_ref, wu1_ref, wu2_ref, bu_ref, g_ref, bt_ref,
         w1_ref, w2_ref, w3_ref, bm_ref) = refs[2 + n_agg:13 + n_agg]
        out_ref, b1_ref, b2_ref = refs[13 + n_agg:]
        h = h_ref[...]
        cnt = jnp.maximum(c0_ref[...] + c1_ref[...], 1.0)
        asum = agg_refs[0][...]
        for r in agg_refs[1:]:
            asum = asum + r[...]
        agg = asum / cnt
        u = h @ wu1_ref[...] + agg @ wu2_ref[...] + bu_ref[...]
        hn = h + _ln(u, g_ref[...], bt_ref[...])
        out_ref[...] = hn
        p = pos_ref[...] @ w3_ref[...]
        b1_ref[...] = hn @ w1_ref[...] + p
        b2_ref[...] = hn @ w2_ref[...] - p + bm_ref[...]

    def call(h, pos, aggs, c0, c1, wu1, wu2, bu, g, bt, w1, w2, w3, bm):
        return pl.pallas_call(
            body,
            grid=(N_NODES // NODE_BLK,),
            in_specs=([node_spec, pos_spec] + [node_spec] * n_agg +
                      [cnt_spec, cnt_spec, wide_spec, wide_spec, vec_spec,
                       vec_spec, vec_spec, wide_spec, wide_spec, w3_spec,
                       vec_spec]),
            out_specs=[node_spec, node_spec, node_spec],
            out_shape=[
                jax.ShapeDtypeStruct((N_NODES, DIM), jnp.float32),
                jax.ShapeDtypeStruct((N_NODES, DIM), jnp.float32),
                jax.ShapeDtypeStruct((N_NODES, DIM), jnp.float32),
            ],
        )(h, pos, *aggs, c0, c1, wu1, wu2, bu, g, bt, w1, w2, w3, bm)

    return call


def _wid():
    return lax.axis_index("s") * NC + lax.axis_index("c")


def _make_gather(cpt):
    e_out = NW * cpt * CHUNK

    def body(b1_hbm, b2_hbm, row_hbm, col_hbm, t_hbm,
             ridx_all, cidx_all, r1, r2, g1sem, g2sem, wsem):
        wid = _wid()
        tile_base = pl.multiple_of(wid * (cpt * CHUNK), CHUNK)
        pltpu.sync_copy(row_hbm.at[pl.ds(wid * cpt, cpt)], ridx_all)
        pltpu.sync_copy(col_hbm.at[pl.ds(wid * cpt, cpt)], cidx_all)

        def issue_gather(s, c):
            pltpu.async_copy(b1_hbm.at[ridx_all.at[c]], r1[s], g1sem[s])
            pltpu.async_copy(b2_hbm.at[cidx_all.at[c]], r2[s], g2sem[s])

        def wait_gather(s):
            pltpu.make_async_copy(
                b1_hbm.at[pl.ds(0, CHUNK)], r1[s], g1sem[s]).wait()
            pltpu.make_async_copy(
                b2_hbm.at[pl.ds(0, CHUNK)], r2[s], g2sem[s]).wait()

        def wait_write(s):
            pltpu.make_async_copy(
                r1[s], t_hbm.at[pl.ds(0, CHUNK)], wsem[s]).wait()

        def add_and_write(s, c):
            r1s, r2s = r1[s], r2[s]

            @plsc.parallel_loop(0, CHUNK, unroll=4)
            def _(j):
                for k in range(DIM // L):
                    sl = pl.ds(k * L, L)
                    r1s[j, sl] = r1s[j, sl] + r2s[j, sl]
            base = pl.multiple_of(tile_base + c * CHUNK, CHUNK)
            pltpu.async_copy(r1[s], t_hbm.at[pl.ds(base, CHUNK)], wsem[s])

        for s in range(3):
            issue_gather(s, s)
        for s in range(3):
            wait_gather(s)
            add_and_write(s, s)
            issue_gather(s, s + 3)

        nb = (cpt - 3) // 3

        def loop_body(i, carry):
            for s in range(3):
                c = 3 * i + s
                wait_gather(s)
                wait_write(s)
                add_and_write(s, c)

                @pl.when(c + 3 < cpt)
                def _():
                    issue_gather(s, c + 3)
            return carry

        lax.fori_loop(1, 1 + nb, loop_body, 0)
        for c in range(3 + 3 * nb, cpt):
            s = c % 3
            wait_gather(s)
            wait_write(s)
            add_and_write(s, c)
        for s in range(3):
            wait_write(s)

    return functools.partial(
        pl.kernel,
        out_type=jax.ShapeDtypeStruct((e_out, DIM), jnp.float32),
        mesh=_SC_MESH,
        scratch_types=[
            pltpu.VMEM((cpt, CHUNK), jnp.int32),
            pltpu.VMEM((cpt, CHUNK), jnp.int32),
            [pltpu.VMEM((CHUNK, DIM), jnp.float32)] * 3,
            [pltpu.VMEM((CHUNK, DIM), jnp.float32)] * 3,
            [pltpu.SemaphoreType.DMA] * 3,
            [pltpu.SemaphoreType.DMA] * 3,
            [pltpu.SemaphoreType.DMA] * 3,
        ],
    )(body)


def _make_scatter(cpt):
    assert cpt % 2 == 0

    def body(m_hbm, col_hbm, zeros_hbm, out_hbm, cidx_all, mv, msem, agg):
        sid = lax.axis_index("s")
        cid = lax.axis_index("c")
        wid = _wid()
        tile_base = pl.multiple_of(wid * (cpt * CHUNK), CHUNK)
        pltpu.sync_copy(col_hbm.at[pl.ds(wid * cpt, cpt)], cidx_all)

        @pl.when(sid == 0)
        def _():
            pltpu.sync_copy(zeros_hbm, agg)

        plsc.subcore_barrier()

        def fetch_m(s, c):
            base = pl.multiple_of(tile_base + c * CHUNK, CHUNK)
            pltpu.async_copy(m_hbm.at[pl.ds(base, CHUNK)], mv[s], msem[s])

        def wait_m(s):
            pltpu.make_async_copy(
                m_hbm.at[pl.ds(0, CHUNK)], mv[s], msem[s]).wait()

        for s in range(2):
            fetch_m(s, s)

        def loop_body(i, carry):
            for s in range(2):
                c = 2 * i + s
                wait_m(s)
                pltpu.sync_copy(mv[s], agg.at[cidx_all.at[c]], add=True)

                @pl.when(c + 2 < cpt)
                def _():
                    fetch_m(s, c + 2)
            return carry

        lax.fori_loop(0, cpt // 2, loop_body, 0)
        plsc.subcore_barrier()

        @pl.when(sid == 0)
        def _():
            pltpu.sync_copy(agg, out_hbm.at[cid])

    return functools.partial(
        pl.kernel,
        out_type=jax.ShapeDtypeStruct((NC, AGG_ROWS, DIM), jnp.float32),
        mesh=_SC_MESH,
        scratch_types=[
            pltpu.VMEM((cpt, CHUNK), jnp.int32),
            [pltpu.VMEM((CHUNK, DIM), jnp.float32)] * 2,
            [pltpu.SemaphoreType.DMA] * 2,
            pltpu.VMEM_SHARED((AGG_ROWS, DIM), jnp.float32),
        ],
    )(body)


SPLITS = (40, 40)
_GATHERS = {c: _make_gather(c) for c in set(SPLITS)}
_SCATTERS = {c: _make_scatter(c) for c in set(SPLITS)}
N_AGG = 2 * len(SPLITS)
_upd = _make_update(N_AGG)
_upd_b12 = _make_update_b12(N_AGG)


def _cnt_body(col_hbm, zeros_hbm, out_hbm, cidx_all, ones_v, cnt):
    sid = lax.axis_index("s")
    cid = lax.axis_index("c")
    wid = _wid()
    for k in range(CHUNK // L):
        ones_v[pl.ds(k * L, L)] = jnp.full((L,), 1.0, jnp.float32)
    pltpu.sync_copy(col_hbm.at[pl.ds(wid * CPT, CPT)], cidx_all)

    @pl.when(sid == 0)
    def _():
        pltpu.sync_copy(zeros_hbm, cnt)

    plsc.subcore_barrier()

    def chunk(i, carry):
        pltpu.sync_copy(ones_v, cnt.at[cidx_all.at[i]], add=True)
        return carry

    lax.fori_loop(0, CPT, chunk, 0)
    plsc.subcore_barrier()

    @pl.when(sid == 0)
    def _():
        pltpu.sync_copy(cnt, out_hbm.at[cid])


@functools.partial(
    pl.kernel,
    out_type=jax.ShapeDtypeStruct((NC, AGG_ROWS), jnp.float32),
    mesh=_SC_MESH,
    scratch_types=[
        pltpu.VMEM((CPT, CHUNK), jnp.int32),
        pltpu.VMEM((CHUNK,), jnp.float32),
        pltpu.VMEM_SHARED((AGG_ROWS,), jnp.float32),
    ],
)
def _cnt_kernel(col_hbm, zeros_hbm, out_hbm, cidx_all, ones_v, cnt):
    _cnt_body(col_hbm, zeros_hbm, out_hbm, cidx_all, ones_v, cnt)


def kernel(x, pos, edge_index, W_msg, b_msg, g_msg, bt_msg, W_upd, b_upd,
           g_upd, bt_upd):
    row = edge_index[0].astype(jnp.int32)
    col = edge_index[1].astype(jnp.int32)
    pad = E_PAD - N_EDGES
    row_g = jnp.pad(row, (0, pad)).reshape(-1, CHUNK)
    col_g = jnp.pad(col, (0, pad)).reshape(-1, CHUNK)
    col_s = jnp.pad(col, (0, pad),
                    constant_values=TRASH).reshape(-1, CHUNK)
    pos_pad = jnp.pad(pos, ((0, 0), (0, 8 - pos.shape[1])))

    zeros2d = jnp.zeros((AGG_ROWS, DIM), jnp.float32)
    zeros1d = jnp.zeros((AGG_ROWS,), jnp.float32)

    cnt_part = _cnt_kernel(col_s, zeros1d)
    c0 = cnt_part[0][:, None]
    c1 = cnt_part[1][:, None]

    starts, accum = [], 0
    for c in SPLITS:
        starts.append(accum)
        accum += c * NW
    row_h = [row_g[s:s + c * NW] for s, c in zip(starts, SPLITS)]
    col_gh = [col_g[s:s + c * NW] for s, c in zip(starts, SPLITS)]
    col_sh = [col_s[s:s + c * NW] for s, c in zip(starts, SPLITS)]

    def msg_w(i):
        return (W_msg[i, :DIM], W_msg[i, DIM:2 * DIM],
                jnp.pad(W_msg[i, 2 * DIM:], ((0, 8 - 3), (0, 0))),
                b_msg[i][None])

    h = x
    w1, w2, w3, bm = msg_w(0)
    b1, b2 = _compute_b12(h, pos_pad, w1, w2, w3, bm)
    for i in range(MP_STEPS):
        aggs = []
        ts = [_GATHERS[c](b1, b2, row_h[k], col_gh[k])
              for k, c in enumerate(SPLITS)]
        for k, c in enumerate(SPLITS):
            m = _edge_mlp(ts[k], g_msg[i][None], bt_msg[i][None])
            agg_part = _SCATTERS[c](m, col_sh[k], zeros2d)
            aggs += [agg_part[0], agg_part[1]]
        if i + 1 < MP_STEPS:
            w1, w2, w3, bm = msg_w(i + 1)
            h, b1, b2 = _upd_b12(h, pos_pad, aggs, c0, c1,
                                 W_upd[i, :DIM], W_upd[i, DIM:],
                                 b_upd[i][None], g_upd[i][None],
                                 bt_upd[i][None], w1, w2, w3, bm)
        else:
            h = _upd(h, aggs, c0, c1,
                     W_upd[i, :DIM], W_upd[i, DIM:], b_upd[i][None],
                     g_upd[i][None], bt_upd[i][None])
    return h
